# Initial kernel scaffold; baseline (speedup 1.0000x reference)
import jax
import jax.numpy as jnp
from jax.experimental import pallas as pl


def kernel(x, edge_index, z, params):
    raise NotImplementedError("write your pallas kernel here")


# speedup vs baseline: 3.6257x; 3.6257x over previous
import functools
import math

import jax
import jax.numpy as jnp
from jax.experimental import pallas as pl
from jax.experimental.pallas import tpu as pltpu

_B = 8192
_IN = 128
_HID = 128
_OUT = 128
_NN = 12
_TB = 256
_RSQ = 1.0 / math.sqrt(128.0)


def _fused_kernel(c_ref, x_ref, w1_ref, b1_ref, w2_ref, b2_ref,
                  z_ref, zw_ref, zb_ref,
                  l1ws_ref, l1wa_ref, l1b_ref,
                  l2ws_ref, l2wa_ref, l2b_ref, o_ref):
    x = x_ref[...]
    h1 = jnp.maximum(x @ w1_ref[...] + b1_ref[...], 0.0)
    outs = []
    for n in range(_NN):
        hn = h1[:, n * _HID:(n + 1) * _HID]
        outs.append(hn @ w2_ref[n * _HID:(n + 1) * _HID, :] + b2_ref[n:n + 1, :])
    h = jnp.concatenate(outs, axis=0)

    zfeat = z_ref[...] @ zw_ref[...] + zb_ref[...]

    def gnn_layer(hs, ws, wa, bvec, zadd, use_relu):
        gp = {}
        for n in range(_NN):
            hn = hs[n * _TB:(n + 1) * _TB, :]
            for m in range(n, _NN):
                hm = hs[m * _TB:(m + 1) * _TB, :]
                gp[(n, m)] = jnp.sum(hn * hm, axis=1, keepdims=True) * _RSQ

        def score(n, m):
            return gp[(n, m) if n <= m else (m, n)]

        aggs = []
        for n in range(_NN):
            cs = [c_ref[n, m] for m in range(_NN)]
            mx = jnp.full((_TB, 1), -jnp.inf, jnp.float32)
            for m in range(_NN):
                mx = jnp.where(cs[m] > 0, jnp.maximum(mx, score(n, m)), mx)
            mx = jnp.where(jnp.isfinite(mx), mx, 0.0)
            exs = []
            den = jnp.zeros((_TB, 1), jnp.float32)
            for m in range(_NN):
                e = jnp.exp(score(n, m) - mx)
                exs.append(e)
                den = den + cs[m] * e
            inv = 1.0 / (den + 1e-9)
            acc = jnp.zeros((_TB, _OUT), jnp.float32)
            for m in range(_NN):
                w = (cs[m] * exs[m]) * inv
                acc = acc + w * hs[m * _TB:(m + 1) * _TB, :]
            aggs.append(acc)
        agg = jnp.concatenate(aggs, axis=0)

        out = hs @ ws + agg @ wa + bvec
        if zadd is not None:
            out = out + zadd
        if use_relu:
            out = jnp.maximum(out, 0.0)
        return out

    h = gnn_layer(h, l1ws_ref[...], l1wa_ref[...], l1b_ref[...], zfeat, True)
    h = gnn_layer(h, l2ws_ref[...], l2wa_ref[...], l2b_ref[...], None, False)

    acc = h[0:_TB, :]
    for n in range(1, _NN):
        acc = acc + h[n * _TB:(n + 1) * _TB, :]
    o_ref[...] = acc * (1.0 / _NN)


def _full(shape):
    zeros = (0,) * len(shape)
    return pl.BlockSpec(shape, lambda i, z=zeros: z)


@jax.jit
def _run(c, x, w1cat, b1cat, w2cat, b2, z2, zw, zb, l1ws, l1wa, l1b, l2ws, l2wa, l2b):
    return pl.pallas_call(
        _fused_kernel,
        grid=(_B // _TB,),
        in_specs=[
            pl.BlockSpec(memory_space=pltpu.SMEM),
            pl.BlockSpec((_TB, _IN), lambda i: (i, 0)),
            _full((_IN, _NN * _HID)),
            _full((1, _NN * _HID)),
            _full((_NN * _HID, _OUT)),
            _full((_NN, _OUT)),
            _full((1, 16)),
            _full((16, _HID)),
            _full((1, _HID)),
            _full((_OUT, _HID)),
            _full((_OUT, _HID)),
            _full((1, _HID)),
            _full((_HID, _OUT)),
            _full((_HID, _OUT)),
            _full((1, _OUT)),
        ],
        out_specs=pl.BlockSpec((_TB, _OUT), lambda i: (i, 0)),
        out_shape=jax.ShapeDtypeStruct((_B, _OUT), jnp.float32),
        compiler_params=pltpu.CompilerParams(
            dimension_semantics=("parallel",)),
    )(c, x, w1cat, b1cat, w2cat, b2, z2, zw, zb, l1ws, l1wa, l1b, l2ws, l2wa, l2b)


def kernel(x, edge_index, z, params):
    src = edge_index[0]
    dst = edge_index[1]
    c = jnp.zeros((_NN, _NN), jnp.float32).at[dst, src].add(1.0)
    w1cat = params["gauge_W1"].transpose(1, 0, 2).reshape(_IN, _NN * _HID)
    b1cat = params["gauge_b1"].reshape(1, _NN * _HID)
    w2cat = params["gauge_W2"].reshape(_NN * _HID, _OUT)
    return _run(
        c, x, w1cat, b1cat, w2cat, params["gauge_b2"],
        z.reshape(1, 16), params["z_W"], params["z_b"].reshape(1, _HID),
        params["l1_Ws"], params["l1_Wa"], params["l1_b"].reshape(1, _HID),
        params["l2_Ws"], params["l2_Wa"], params["l2_b"].reshape(1, _OUT),
    )
